# Initial kernel scaffold; baseline (speedup 1.0000x reference)
import jax
import jax.numpy as jnp
from jax.experimental import pallas as pl


def kernel(msg, index, t, dim_size):
    raise NotImplementedError("write your pallas kernel here")


# speedup vs baseline: 4.6965x; 4.6965x over previous
import functools

import jax
import jax.numpy as jnp
from jax import lax
from jax.experimental import pallas as pl
from jax.experimental.pallas import tpu as pltpu
from jax.experimental.pallas import tpu_sc as plsc

N_EDGES = 320000
D = 128
N_NODES = 10000
N_PAD = 10240
NC = 2
NS = 16
L = 16
NW = NC * NS
EPT = N_EDGES // NW
B = 80
NCHUNK = EPT // B
RPT = N_PAD // NS
R2 = N_PAD // NW

_mesh = plsc.VectorSubcoreMesh(core_axis_name="c", subcore_axis_name="s")


def _zero_vmem(ref, nwords):
    def body(j, _):
        ref[pl.ds(j * L, L)] = jnp.zeros((L,), jnp.float32)
        return 0
    lax.fori_loop(0, nwords // L, body, 0)


@functools.partial(
    pl.kernel,
    out_type=(
        jax.ShapeDtypeStruct((NC, N_PAD, D), jnp.float32),
        jax.ShapeDtypeStruct((NC * N_PAD,), jnp.float32),
    ),
    mesh=_mesh,
    scratch_types=[
        pltpu.VMEM_SHARED((N_PAD, D), jnp.float32),
        pltpu.VMEM_SHARED((N_PAD,), jnp.float32),
        pltpu.VMEM((B,), jnp.int32),
        pltpu.VMEM((B, D), jnp.float32),
        pltpu.VMEM((B,), jnp.float32),
        pltpu.VMEM((RPT,), jnp.float32),
    ],
)
def _scatter_stage(msg_hbm, idx_hbm, psum_hbm, pcnt_hbm,
                   acc_sh, cnt_sh, idx_v, rows_v, ones_v, zvec_v):
    cid = lax.axis_index("c")
    sid = lax.axis_index("s")
    wid = cid * NS + sid

    def zrow(r, _):
        def zcol(j, _):
            rows_v[r, pl.ds(j * L, L)] = jnp.zeros((L,), jnp.float32)
            return 0
        lax.fori_loop(0, D // L, zcol, 0)
        return 0
    lax.fori_loop(0, B, zrow, 0)
    _zero_vmem(zvec_v, RPT)

    def one(j, _):
        ones_v[pl.ds(j * L, L)] = jnp.ones((L,), jnp.float32)
        return 0
    lax.fori_loop(0, B // L, one, 0)

    base_r = sid * RPT
    for k in range(RPT // B):
        pltpu.sync_copy(rows_v, acc_sh.at[pl.ds(base_r + k * B, B), :])
    pltpu.sync_copy(zvec_v, cnt_sh.at[pl.ds(base_r, RPT)])
    plsc.subcore_barrier()

    ebase = wid * EPT

    def chunk(c, _):
        off = ebase + c * B
        pltpu.sync_copy(idx_hbm.at[pl.ds(off, B)], idx_v)
        pltpu.sync_copy(msg_hbm.at[pl.ds(off, B), :], rows_v)
        pltpu.sync_copy(rows_v, acc_sh.at[idx_v], add=True)
        pltpu.sync_copy(ones_v, cnt_sh.at[idx_v], add=True)
        return 0
    lax.fori_loop(0, NCHUNK, chunk, 0)
    plsc.subcore_barrier()

    pltpu.sync_copy(acc_sh.at[pl.ds(base_r, RPT), :],
                    psum_hbm.at[cid, pl.ds(base_r, RPT), :])
    pltpu.sync_copy(cnt_sh.at[pl.ds(base_r, RPT)],
                    pcnt_hbm.at[pl.ds(cid * N_PAD + base_r, RPT)])


@functools.partial(
    pl.kernel,
    out_type=jax.ShapeDtypeStruct((N_PAD, D), jnp.float32),
    mesh=_mesh,
    scratch_types=[
        pltpu.VMEM((R2, D), jnp.float32),
        pltpu.VMEM((R2, D), jnp.float32),
        pltpu.VMEM((R2,), jnp.float32),
        pltpu.VMEM((R2,), jnp.float32),
        pltpu.VMEM((R2 + L,), jnp.float32),
    ],
)
def _combine_stage(psum_hbm, pcnt_hbm, out_hbm, pa, pb, ca, cb, rcp):
    cid = lax.axis_index("c")
    sid = lax.axis_index("s")
    wid = cid * NS + sid
    base = wid * R2

    pltpu.sync_copy(psum_hbm.at[0, pl.ds(base, R2), :], pa)
    pltpu.sync_copy(psum_hbm.at[1, pl.ds(base, R2), :], pb)
    pltpu.sync_copy(pcnt_hbm.at[pl.ds(base, R2)], ca)
    pltpu.sync_copy(pcnt_hbm.at[pl.ds(N_PAD + base, R2)], cb)

    def recip(i, _):
        c = ca[pl.ds(i * L, L)] + cb[pl.ds(i * L, L)]
        rcp[pl.ds(i * L, L)] = 1.0 / jnp.maximum(c, 1.0)
        return 0
    lax.fori_loop(0, R2 // L, recip, 0)
    rcp[pl.ds(R2, L)] = jnp.ones((L,), jnp.float32)

    def row(r, _):
        s = rcp[pl.ds(r, L)][0]
        def col(j, _):
            pa[r, pl.ds(j * L, L)] = (
                pa[r, pl.ds(j * L, L)] + pb[r, pl.ds(j * L, L)]) * s
            return 0
        lax.fori_loop(0, D // L, col, 0)
        return 0
    lax.fori_loop(0, R2, row, 0)

    pltpu.sync_copy(pa, out_hbm.at[pl.ds(base, R2), :])


def kernel(msg, index, t, dim_size):
    del t, dim_size
    idx32 = index.astype(jnp.int32)
    psum, pcnt = _scatter_stage(msg, idx32)
    out = _combine_stage(psum, pcnt)
    return out[:N_NODES]


# speedup vs baseline: 7.4796x; 1.5926x over previous
import functools

import jax
import jax.numpy as jnp
from jax import lax
from jax.experimental import pallas as pl
from jax.experimental.pallas import tpu as pltpu
from jax.experimental.pallas import tpu_sc as plsc

N_EDGES = 320000
D = 128
N_NODES = 10000
N_PAD = 10240
NC = 2
NS = 16
L = 16
NW = NC * NS
EPT = N_EDGES // NW
B = 80
NCHUNK = EPT // B
RPT = N_PAD // NS
R2 = N_PAD // NW

_mesh = plsc.VectorSubcoreMesh(core_axis_name="c", subcore_axis_name="s")


def _zero_vmem(ref, nwords):
    def body(j, _):
        ref[pl.ds(j * L, L)] = jnp.zeros((L,), jnp.float32)
        return 0
    lax.fori_loop(0, nwords // L, body, 0)


@functools.partial(
    pl.kernel,
    out_type=(
        jax.ShapeDtypeStruct((NC, N_PAD, D), jnp.float32),
        jax.ShapeDtypeStruct((NC * N_PAD,), jnp.float32),
    ),
    mesh=_mesh,
    scratch_types=[
        pltpu.VMEM_SHARED((N_PAD, D), jnp.float32),
        pltpu.VMEM_SHARED((N_PAD,), jnp.float32),
        pltpu.VMEM((NCHUNK, B), jnp.int32),
        pltpu.VMEM((B, D), jnp.float32),
        pltpu.VMEM((B, D), jnp.float32),
        pltpu.VMEM((B,), jnp.float32),
        pltpu.VMEM((RPT,), jnp.float32),
        pltpu.SemaphoreType.DMA,
        pltpu.SemaphoreType.DMA,
        pltpu.SemaphoreType.DMA,
        pltpu.SemaphoreType.DMA,
        pltpu.SemaphoreType.DMA,
    ],
)
def _scatter_stage(msg_hbm, idx3_hbm, psum_hbm, pcnt_hbm,
                   acc_sh, cnt_sh, idx_all, rows0, rows1, ones_v, zvec_v,
                   fsem0, fsem1, ssem0, ssem1, csem):
    cid = lax.axis_index("c")
    sid = lax.axis_index("s")
    wid = cid * NS + sid
    ebase = wid * EPT

    def zrow(r, _):
        def zcol(j, _):
            rows0[r, pl.ds(j * L, L)] = jnp.zeros((L,), jnp.float32)
            return 0
        lax.fori_loop(0, D // L, zcol, 0)
        return 0
    lax.fori_loop(0, B, zrow, 0)
    _zero_vmem(zvec_v, RPT)

    def one(j, _):
        ones_v[pl.ds(j * L, L)] = jnp.ones((L,), jnp.float32)
        return 0
    lax.fori_loop(0, B // L, one, 0)

    pltpu.sync_copy(idx3_hbm.at[wid], idx_all)

    base_r = sid * RPT
    for k in range(RPT // B):
        pltpu.sync_copy(rows0, acc_sh.at[pl.ds(base_r + k * B, B), :])
    pltpu.sync_copy(zvec_v, cnt_sh.at[pl.ds(base_r, RPT)])
    plsc.subcore_barrier()

    def fetch_start(c, buf, sem):
        pltpu.async_copy(msg_hbm.at[pl.ds(ebase + c * B, B), :], buf, sem)

    def fetch_wait(c, buf, sem):
        pltpu.make_async_copy(
            msg_hbm.at[pl.ds(ebase + c * B, B), :], buf, sem).wait()

    def scat_start(c, buf, sem):
        pltpu.async_copy(buf, acc_sh.at[idx_all.at[c]], sem, add=True)
        pltpu.async_copy(ones_v, cnt_sh.at[idx_all.at[c]], csem, add=True)

    def scat_wait(c, buf, sem):
        pltpu.make_async_copy(buf, acc_sh.at[idx_all.at[c]], sem).wait()

    fetch_start(0, rows0, fsem0)

    def pair(g, _):
        c0 = 2 * g
        c1 = c0 + 1
        fetch_wait(c0, rows0, fsem0)

        @pl.when(g > 0)
        def _():
            scat_wait(c0 - 1, rows1, ssem1)
        fetch_start(c1, rows1, fsem1)
        scat_start(c0, rows0, ssem0)
        fetch_wait(c1, rows1, fsem1)
        scat_wait(c0, rows0, ssem0)
        fetch_start(c0 + 2, rows0, fsem0)
        scat_start(c1, rows1, ssem1)
        return 0
    lax.fori_loop(0, (NCHUNK - 1) // 2, pair, 0)

    scat_wait(NCHUNK - 2, rows1, ssem1)
    fetch_wait(NCHUNK - 1, rows0, fsem0)
    pltpu.sync_copy(rows0, acc_sh.at[idx_all.at[NCHUNK - 1]], add=True)
    pltpu.sync_copy(ones_v, cnt_sh.at[idx_all.at[NCHUNK - 1]], add=True)

    def drain(i, _):
        pltpu.make_async_copy(ones_v, cnt_sh.at[idx_all.at[0]], csem).wait()
        return 0
    lax.fori_loop(0, NCHUNK - 1, drain, 0)
    plsc.subcore_barrier()

    pltpu.sync_copy(acc_sh.at[pl.ds(base_r, RPT), :],
                    psum_hbm.at[cid, pl.ds(base_r, RPT), :])
    pltpu.sync_copy(cnt_sh.at[pl.ds(base_r, RPT)],
                    pcnt_hbm.at[pl.ds(cid * N_PAD + base_r, RPT)])


@functools.partial(
    pl.kernel,
    out_type=jax.ShapeDtypeStruct((N_PAD, D), jnp.float32),
    mesh=_mesh,
    scratch_types=[
        pltpu.VMEM((R2, D), jnp.float32),
        pltpu.VMEM((R2, D), jnp.float32),
        pltpu.VMEM((R2,), jnp.float32),
        pltpu.VMEM((R2,), jnp.float32),
        pltpu.VMEM((R2 + L,), jnp.float32),
    ],
)
def _combine_stage(psum_hbm, pcnt_hbm, out_hbm, pa, pb, ca, cb, rcp):
    cid = lax.axis_index("c")
    sid = lax.axis_index("s")
    wid = cid * NS + sid
    base = wid * R2

    pltpu.sync_copy(psum_hbm.at[0, pl.ds(base, R2), :], pa)
    pltpu.sync_copy(psum_hbm.at[1, pl.ds(base, R2), :], pb)
    pltpu.sync_copy(pcnt_hbm.at[pl.ds(base, R2)], ca)
    pltpu.sync_copy(pcnt_hbm.at[pl.ds(N_PAD + base, R2)], cb)

    def recip(i, _):
        c = ca[pl.ds(i * L, L)] + cb[pl.ds(i * L, L)]
        rcp[pl.ds(i * L, L)] = 1.0 / jnp.maximum(c, 1.0)
        return 0
    lax.fori_loop(0, R2 // L, recip, 0)
    rcp[pl.ds(R2, L)] = jnp.ones((L,), jnp.float32)

    def row(r, _):
        s = rcp[pl.ds(r, L)][0]
        def col(j, _):
            pa[r, pl.ds(j * L, L)] = (
                pa[r, pl.ds(j * L, L)] + pb[r, pl.ds(j * L, L)]) * s
            return 0
        lax.fori_loop(0, D // L, col, 0)
        return 0
    lax.fori_loop(0, R2, row, 0)

    pltpu.sync_copy(pa, out_hbm.at[pl.ds(base, R2), :])


def kernel(msg, index, t, dim_size):
    del t, dim_size
    idx32 = index.astype(jnp.int32).reshape(NW, NCHUNK, B)
    psum, pcnt = _scatter_stage(msg, idx32)
    out = _combine_stage(psum, pcnt)
    return out[:N_NODES]


# speedup vs baseline: 10.8028x; 1.4443x over previous
import functools

import jax
import jax.numpy as jnp
from jax import lax
from jax.experimental import pallas as pl
from jax.experimental.pallas import tpu as pltpu
from jax.experimental.pallas import tpu_sc as plsc

N_EDGES = 320000
D = 128
N_NODES = 10000
N_PAD = 10240
NC = 2
NS = 16
L = 16
NW = NC * NS
EPT = N_EDGES // NW
B = 80
NCHUNK = EPT // B
NBUF = 4
RPT = N_PAD // NS
R2 = N_PAD // NW

_mesh = plsc.VectorSubcoreMesh(core_axis_name="c", subcore_axis_name="s")


def _zero_vmem(ref, nwords):
    def body(j, _):
        ref[pl.ds(j * L, L)] = jnp.zeros((L,), jnp.float32)
        return 0
    lax.fori_loop(0, nwords // L, body, 0)


@functools.partial(
    pl.kernel,
    out_type=(
        jax.ShapeDtypeStruct((NC, N_PAD, D), jnp.float32),
        jax.ShapeDtypeStruct((NC * N_PAD,), jnp.float32),
    ),
    mesh=_mesh,
    scratch_types=[
        pltpu.VMEM_SHARED((N_PAD, D), jnp.float32),
        pltpu.VMEM_SHARED((N_PAD,), jnp.float32),
        pltpu.VMEM((NBUF, B), jnp.int32),
        pltpu.VMEM((B, D), jnp.float32),
        pltpu.VMEM((B, D), jnp.float32),
        pltpu.VMEM((B, D), jnp.float32),
        pltpu.VMEM((B, D), jnp.float32),
        pltpu.VMEM((B,), jnp.float32),
        pltpu.VMEM((RPT,), jnp.float32),
        pltpu.SemaphoreType.DMA,
        pltpu.SemaphoreType.DMA,
        pltpu.SemaphoreType.DMA,
        pltpu.SemaphoreType.DMA,
        pltpu.SemaphoreType.DMA,
        pltpu.SemaphoreType.DMA,
    ],
)
def _scatter_stage(msg_hbm, idx_hbm, psum_hbm, pcnt_hbm,
                   acc_sh, cnt_sh, idx_ring, rows0, rows1, rows2, rows3,
                   ones_v, zvec_v, fsem0, fsem1, fsem2, fsem3, ssem, csem):
    cid = lax.axis_index("c")
    sid = lax.axis_index("s")
    wid = cid * NS + sid
    ebase = wid * EPT
    bufs = (rows0, rows1, rows2, rows3)
    fsems = (fsem0, fsem1, fsem2, fsem3)

    def zrow(r, _):
        def zcol(j, _):
            rows0[r, pl.ds(j * L, L)] = jnp.zeros((L,), jnp.float32)
            return 0
        lax.fori_loop(0, D // L, zcol, 0)
        return 0
    lax.fori_loop(0, B, zrow, 0)
    _zero_vmem(zvec_v, RPT)

    def one(j, _):
        ones_v[pl.ds(j * L, L)] = jnp.ones((L,), jnp.float32)
        return 0
    lax.fori_loop(0, B // L, one, 0)

    base_r = sid * RPT
    for k in range(RPT // B):
        pltpu.sync_copy(rows0, acc_sh.at[pl.ds(base_r + k * B, B), :])
    pltpu.sync_copy(zvec_v, cnt_sh.at[pl.ds(base_r, RPT)])
    plsc.subcore_barrier()

    def fetch_start(c, k):
        pltpu.async_copy(msg_hbm.at[pl.ds(ebase + c * B, B), :],
                         bufs[k], fsems[k])
        pltpu.async_copy(idx_hbm.at[pl.ds(ebase + c * B, B)],
                         idx_ring.at[k], fsems[k])

    def fetch_wait(c, k):
        pltpu.make_async_copy(msg_hbm.at[pl.ds(ebase + c * B, B), :],
                              bufs[k], fsems[k]).wait()
        pltpu.make_async_copy(idx_hbm.at[pl.ds(ebase + c * B, B)],
                              idx_ring.at[k], fsems[k]).wait()

    def scat(c, k):
        pltpu.async_copy(bufs[k], acc_sh.at[idx_ring.at[k]], ssem, add=True)
        pltpu.async_copy(ones_v, cnt_sh.at[idx_ring.at[k]], csem, add=True)
        pltpu.make_async_copy(bufs[k], acc_sh.at[idx_ring.at[k]], ssem).wait()

    for k in range(NBUF):
        fetch_start(k, k)

    def quad(g, _):
        for k in range(NBUF):
            c = NBUF * g + k
            fetch_wait(c, k)
            scat(c, k)

            def refill(c=c, k=k):
                fetch_start(c + NBUF, k)
            pl.when(c + NBUF <= NCHUNK - 1)(refill)
        return 0
    lax.fori_loop(0, (NCHUNK - 1) // NBUF, quad, 0)

    fetch_wait(NCHUNK - 1, 0)
    scat(NCHUNK - 1, 0)

    def drain(i, _):
        pltpu.make_async_copy(ones_v, cnt_sh.at[idx_ring.at[0]], csem).wait()
        return 0
    lax.fori_loop(0, NCHUNK, drain, 0)
    plsc.subcore_barrier()

    pltpu.sync_copy(acc_sh.at[pl.ds(base_r, RPT), :],
                    psum_hbm.at[cid, pl.ds(base_r, RPT), :])
    pltpu.sync_copy(cnt_sh.at[pl.ds(base_r, RPT)],
                    pcnt_hbm.at[pl.ds(cid * N_PAD + base_r, RPT)])


@functools.partial(
    pl.kernel,
    out_type=jax.ShapeDtypeStruct((N_PAD, D), jnp.float32),
    mesh=_mesh,
    scratch_types=[
        pltpu.VMEM((R2, D), jnp.float32),
        pltpu.VMEM((R2, D), jnp.float32),
        pltpu.VMEM((R2,), jnp.float32),
        pltpu.VMEM((R2,), jnp.float32),
        pltpu.VMEM((R2 + L,), jnp.float32),
    ],
)
def _combine_stage(psum_hbm, pcnt_hbm, out_hbm, pa, pb, ca, cb, rcp):
    cid = lax.axis_index("c")
    sid = lax.axis_index("s")
    wid = cid * NS + sid
    base = wid * R2

    pltpu.sync_copy(psum_hbm.at[0, pl.ds(base, R2), :], pa)
    pltpu.sync_copy(psum_hbm.at[1, pl.ds(base, R2), :], pb)
    pltpu.sync_copy(pcnt_hbm.at[pl.ds(base, R2)], ca)
    pltpu.sync_copy(pcnt_hbm.at[pl.ds(N_PAD + base, R2)], cb)

    def recip(i, _):
        c = ca[pl.ds(i * L, L)] + cb[pl.ds(i * L, L)]
        rcp[pl.ds(i * L, L)] = 1.0 / jnp.maximum(c, 1.0)
        return 0
    lax.fori_loop(0, R2 // L, recip, 0)
    rcp[pl.ds(R2, L)] = jnp.ones((L,), jnp.float32)

    def row(r, _):
        s = rcp[pl.ds(r, L)][0]
        def col(j, _):
            pa[r, pl.ds(j * L, L)] = (
                pa[r, pl.ds(j * L, L)] + pb[r, pl.ds(j * L, L)]) * s
            return 0
        lax.fori_loop(0, D // L, col, 0)
        return 0
    lax.fori_loop(0, R2, row, 0)

    pltpu.sync_copy(pa, out_hbm.at[pl.ds(base, R2), :])


def kernel(msg, index, t, dim_size):
    del t, dim_size
    idx32 = index.astype(jnp.int32)
    psum, pcnt = _scatter_stage(msg, idx32)
    out = _combine_stage(psum, pcnt)
    return out[:N_NODES]
